# Initial kernel scaffold; baseline (speedup 1.0000x reference)
import jax
import jax.numpy as jnp
from jax.experimental import pallas as pl


def kernel(pos_3d, temperature, parent_choice, hard, centers, log_radii):
    raise NotImplementedError("write your pallas kernel here")


# speedup vs baseline: 1.5284x; 1.5284x over previous
import jax
import jax.numpy as jnp
from jax.experimental import pallas as pl

_N_SPHERES = 64
_TOTAL = 512
_ROWS = 1024


def _router_body(s_ref, pos_ref, pc_ref, ct_ref, probs_ref, choice_ref):
    s = s_ref[...]
    dx = pos_ref[:, 0:1] - ct_ref[0:1, :]
    dy = pos_ref[:, 1:2] - ct_ref[1:2, :]
    dz = pos_ref[:, 2:3] - ct_ref[2:3, :]
    d_sq = (dx * dx + dy * dy) + dz * dz
    logits = (-d_sq) / s

    lpc = jnp.log(pc_ref[...] + 1e-10)
    logits = jnp.concatenate(
        [logits[:, g * _N_SPHERES:(g + 1) * _N_SPHERES] + lpc[:, g:g + 1]
         for g in range(8)], axis=1)

    lane = jax.lax.broadcasted_iota(jnp.int32, logits.shape, 1)
    m = jnp.max(logits, axis=-1, keepdims=True)
    cand = jnp.where(logits == m, lane, _TOTAL)
    choice = jnp.min(cand, axis=-1, keepdims=True)
    probs_ref[...] = (lane == choice).astype(jnp.float32)
    choice_ref[...] = choice


def kernel(pos_3d, temperature, parent_choice, hard, centers, log_radii):
    del hard, log_radii
    b = pos_3d.shape[0]
    s = (2.0 * temperature**2 + 1e-8).reshape(1, 1).astype(jnp.float32)
    ct = centers.T
    grid = (b // _ROWS,)
    probs, choice = pl.pallas_call(
        _router_body,
        grid=grid,
        in_specs=[
            pl.BlockSpec((1, 1), lambda i: (0, 0)),
            pl.BlockSpec((_ROWS, 3), lambda i: (i, 0)),
            pl.BlockSpec((_ROWS, 8), lambda i: (i, 0)),
            pl.BlockSpec((3, _TOTAL), lambda i: (0, 0)),
        ],
        out_specs=[
            pl.BlockSpec((_ROWS, _TOTAL), lambda i: (i, 0)),
            pl.BlockSpec((_ROWS, 1), lambda i: (i, 0)),
        ],
        out_shape=[
            jax.ShapeDtypeStruct((b, _TOTAL), jnp.float32),
            jax.ShapeDtypeStruct((b, 1), jnp.int32),
        ],
    )(s, pos_3d, parent_choice, ct)
    return probs, choice.reshape(b)
